# baseline (speedup 1.0000x reference)
import functools

import jax
import jax.numpy as jnp
from jax import lax
from jax.experimental import pallas as pl
from jax.experimental.pallas import tpu as pltpu
from jax.experimental.pallas import tpu_sc as plsc

_NSEG = 2


def _edge_repack(edge_index):
    E = edge_index.shape[1]
    ER = E // 128

    def body(ei_ref, s_ref, d_ref):
        s_ref[...] = ei_ref[0].reshape(ER, 128)
        d_ref[...] = ei_ref[1].reshape(ER, 128)

    src2, dst2 = pl.pallas_call(
        body,
        out_shape=[jax.ShapeDtypeStruct((ER, 128), jnp.int32),
                   jax.ShapeDtypeStruct((ER, 128), jnp.int32)],
    )(edge_index)
    return src2.reshape(E), dst2.reshape(E)


def _edge_matmul(edge_attr, We, soff, ES):
    D = We.shape[1]
    DE = We.shape[0]
    BE = 3200

    def body(ea_ref, we_ref, z_ref):
        z_ref[...] = jnp.dot(ea_ref[...], we_ref[...],
                             preferred_element_type=jnp.float32)

    return pl.pallas_call(
        body,
        grid=(ES // BE,),
        in_specs=[pl.BlockSpec((BE, DE), lambda i: (soff // BE + i, 0)),
                  pl.BlockSpec((DE, D), lambda i: (0, 0))],
        out_specs=pl.BlockSpec((BE, D), lambda i: (i, 0)),
        out_shape=jax.ShapeDtypeStruct((ES, D), jnp.float32),
    )(edge_attr, We)


def _sc_aggregate(x, src, dst, z, soff, ES):
    N, D = x.shape
    info = plsc.get_sparse_core_info()
    NC, NS = info.num_cores, info.num_subcores
    NW = NC * NS
    EW = ES // NW
    C = 40
    NCH = EW // C
    NBUF = 3
    NSB = 2
    NI = 8
    AHEAD = 4
    NL = D // 16
    RB = 40
    NP = ((N + RB * NS - 1) // (RB * NS)) * (RB * NS)
    RN = NP // NS
    NRB = RN // RB
    mesh = plsc.VectorSubcoreMesh(core_axis_name="c", subcore_axis_name="s")

    @functools.partial(
        pl.kernel,
        out_type=jax.ShapeDtypeStruct((NC, NP, D), jnp.float32),
        mesh=mesh,
        scratch_types=[
            pltpu.VMEM((NI, 2, C), jnp.int32),
            pltpu.VMEM((NBUF, C, D), jnp.float32),
            pltpu.VMEM((NBUF, C, D), jnp.float32),
            pltpu.VMEM((NSB, C, D), jnp.float32),
            pltpu.VMEM_SHARED((NP, D), jnp.float32),
            pltpu.SemaphoreType.DMA((NI,)),
            pltpu.SemaphoreType.DMA((NBUF,)),
            pltpu.SemaphoreType.DMA((NBUF,)),
            pltpu.SemaphoreType.DMA((NSB,)),
        ],
    )
    def agg_kernel(x_hbm, src_hbm, dst_hbm, z_hbm, out_hbm,
                   idxr, rows, zrows, mbuf, agg_sh, isem, gsem, zsem, ssem):
        cid = lax.axis_index("c")
        sid = lax.axis_index("s")
        wid = sid * NC + cid
        r0 = sid * RN
        ebase = wid * EW
        gbase = soff + wid * EW

        def zero_row(i, carry):
            for k in range(NL):
                mbuf[0, i, pl.ds(k * 16, 16)] = jnp.zeros((16,), jnp.float32)
            return carry
        lax.fori_loop(0, RB, zero_row, 0)
        for j in range(NRB):
            pltpu.sync_copy(mbuf.at[0], agg_sh.at[pl.ds(r0 + j * RB, RB)])
        plsc.subcore_barrier()

        def issue_idx(c, slot):
            g0 = pl.multiple_of(gbase + c * C, 8)
            pltpu.async_copy(src_hbm.at[pl.ds(g0, C)],
                             idxr.at[slot, 0], isem.at[slot])
            pltpu.async_copy(dst_hbm.at[pl.ds(g0, C)],
                             idxr.at[slot, 1], isem.at[slot])

        def wait_idx(c, slot):
            g0 = pl.multiple_of(gbase + c * C, 8)
            pltpu.make_async_copy(src_hbm.at[pl.ds(g0, C)],
                                  idxr.at[slot, 0], isem.at[slot]).wait()
            pltpu.make_async_copy(dst_hbm.at[pl.ds(g0, C)],
                                  idxr.at[slot, 1], isem.at[slot]).wait()

        def issue_gather(c, b, slot):
            e0 = pl.multiple_of(ebase + c * C, 8)
            pltpu.async_copy(x_hbm.at[idxr.at[slot, 0]], rows.at[b],
                             gsem.at[b])
            pltpu.async_copy(z_hbm.at[pl.ds(e0, C)], zrows.at[b], zsem.at[b])

        for j in range(AHEAD):
            issue_idx(j, j)
        for j in range(NBUF):
            wait_idx(j, j)
            issue_gather(j, j, j)

        def chunk_body(c, carry):
            b = lax.rem(c, NBUF)
            sb = lax.rem(c, NSB)
            ib = lax.rem(c, NI)
            e0 = pl.multiple_of(ebase + c * C, 8)
            pltpu.make_async_copy(x_hbm.at[idxr.at[ib, 0]], rows.at[b],
                                  gsem.at[b]).wait()
            pltpu.make_async_copy(z_hbm.at[pl.ds(e0, C)], zrows.at[b],
                                  zsem.at[b]).wait()

            @pl.when(c >= NSB)
            def _wait_scatter():
                pltpu.make_async_copy(mbuf.at[sb], agg_sh.at[idxr.at[ib, 1]],
                                      ssem.at[sb]).wait()

            @plsc.parallel_loop(0, C, unroll=4)
            def row_body(i):
                for k in range(NL):
                    sl = pl.ds(k * 16, 16)
                    mbuf[sb, i, sl] = jnp.maximum(
                        rows[b, i, sl] + zrows[b, i, sl], 0.0)

            pltpu.async_copy(mbuf.at[sb], agg_sh.at[idxr.at[ib, 1]],
                             ssem.at[sb], add=True)

            @pl.when(c + AHEAD < NCH)
            def _prefetch_idx():
                issue_idx(c + AHEAD, lax.rem(c + AHEAD, NI))

            @pl.when(c + NBUF < NCH)
            def _prefetch_rows():
                cn = c + NBUF
                slot = lax.rem(cn, NI)
                wait_idx(cn, slot)
                issue_gather(cn, b, slot)
            return carry
        lax.fori_loop(0, NCH, chunk_body, 0)

        for j in range(NSB):
            c = NCH - NSB + j
            pltpu.make_async_copy(mbuf.at[c % NSB],
                                  agg_sh.at[idxr.at[c % NI, 1]],
                                  ssem.at[c % NSB]).wait()

        plsc.subcore_barrier()
        for j in range(NRB):
            pltpu.sync_copy(agg_sh.at[pl.ds(r0 + j * RB, RB)], mbuf.at[0])
            pltpu.sync_copy(mbuf.at[0], out_hbm.at[cid, pl.ds(r0 + j * RB, RB)])

    return agg_kernel(x, src, dst, z)


def _mlp(x, parts, W1, b1, W2, b2):
    N, D = x.shape
    NC = parts[0].shape[0]
    BN = 2000

    def body(x_ref, *rest):
        part_refs = rest[:-5]
        w1_ref, b1_ref, w2_ref, b2_ref, o_ref = rest[-5:]
        h = x_ref[...]
        for p_ref in part_refs:
            for c in range(NC):
                h = h + p_ref[c]
        h = jnp.maximum(
            jnp.dot(h, w1_ref[...], preferred_element_type=jnp.float32)
            + b1_ref[...], 0.0)
        o_ref[...] = jnp.dot(h, w2_ref[...],
                             preferred_element_type=jnp.float32) + b2_ref[...]

    part_specs = [pl.BlockSpec((NC, BN, D), lambda i: (0, i, 0))
                  for _ in parts]
    return pl.pallas_call(
        body,
        grid=(N // BN,),
        in_specs=[pl.BlockSpec((BN, D), lambda i: (i, 0))] + part_specs + [
            pl.BlockSpec((D, D), lambda i: (0, 0)),
            pl.BlockSpec((1, D), lambda i: (0, 0)),
            pl.BlockSpec((D, D), lambda i: (0, 0)),
            pl.BlockSpec((1, D), lambda i: (0, 0))],
        out_specs=pl.BlockSpec((BN, D), lambda i: (i, 0)),
        out_shape=jax.ShapeDtypeStruct((N, D), jnp.float32),
    )(x, *parts, W1, b1.reshape(1, D), W2, b2.reshape(1, D))


def kernel(x, edge_index, edge_attr, batch_idx, We, W1, b1, W2, b2):
    E = edge_index.shape[1]
    ES = E // _NSEG
    src, dst = _edge_repack(edge_index)
    parts = []
    for s in range(_NSEG):
        z = _edge_matmul(edge_attr, We, s * ES, ES)
        parts.append(_sc_aggregate(x, src, dst, z, s * ES, ES))
    return _mlp(x, parts, W1, b1, W2, b2)


# speedup vs baseline: 1.3883x; 1.3883x over previous
import functools

import jax
import jax.numpy as jnp
from jax import lax
from jax.experimental import pallas as pl
from jax.experimental.pallas import tpu as pltpu
from jax.experimental.pallas import tpu_sc as plsc

_NSEG = 2


def _edge_repack(edge_index):
    E = edge_index.shape[1]
    ER = E // 128

    def body(ei_ref, s_ref, d_ref):
        s_ref[...] = ei_ref[0].reshape(ER, 128)
        d_ref[...] = ei_ref[1].reshape(ER, 128)

    src2, dst2 = pl.pallas_call(
        body,
        out_shape=[jax.ShapeDtypeStruct((ER, 128), jnp.int32),
                   jax.ShapeDtypeStruct((ER, 128), jnp.int32)],
    )(edge_index)
    return src2.reshape(E), dst2.reshape(E)


def _edge_matmul(edge_attr_t, We, soff, ES):
    D = We.shape[1]
    DE = We.shape[0]
    BE = 3200

    def body(ea_ref, we_ref, z_ref):
        z_ref[...] = lax.dot_general(
            ea_ref[...], we_ref[...],
            dimension_numbers=(((0,), (0,)), ((), ())),
            preferred_element_type=jnp.float32)

    return pl.pallas_call(
        body,
        grid=(ES // BE,),
        in_specs=[pl.BlockSpec((DE, BE), lambda i: (0, soff // BE + i)),
                  pl.BlockSpec((DE, D), lambda i: (0, 0))],
        out_specs=pl.BlockSpec((BE, D), lambda i: (i, 0)),
        out_shape=jax.ShapeDtypeStruct((ES, D), jnp.float32),
    )(edge_attr_t, We)


def _sc_aggregate(x, src, dst, z, soff, ES):
    N, D = x.shape
    info = plsc.get_sparse_core_info()
    NC, NS = info.num_cores, info.num_subcores
    NW = NC * NS
    EW = ES // NW
    C = 40
    NCH = EW // C
    NBUF = 3
    NSB = 2
    NI = 8
    AHEAD = 4
    NL = D // 16
    RB = 40
    NP = ((N + RB * NS - 1) // (RB * NS)) * (RB * NS)
    RN = NP // NS
    NRB = RN // RB
    mesh = plsc.VectorSubcoreMesh(core_axis_name="c", subcore_axis_name="s")

    @functools.partial(
        pl.kernel,
        out_type=jax.ShapeDtypeStruct((NC, NP, D), jnp.float32),
        mesh=mesh,
        scratch_types=[
            pltpu.VMEM((NI, 2, C), jnp.int32),
            pltpu.VMEM((NBUF, C, D), jnp.float32),
            pltpu.VMEM((NBUF, C, D), jnp.float32),
            pltpu.VMEM((NSB, C, D), jnp.float32),
            pltpu.VMEM_SHARED((NP, D), jnp.float32),
            pltpu.SemaphoreType.DMA((NI,)),
            pltpu.SemaphoreType.DMA((NBUF,)),
            pltpu.SemaphoreType.DMA((NBUF,)),
            pltpu.SemaphoreType.DMA((NSB,)),
        ],
    )
    def agg_kernel(x_hbm, src_hbm, dst_hbm, z_hbm, out_hbm,
                   idxr, rows, zrows, mbuf, agg_sh, isem, gsem, zsem, ssem):
        cid = lax.axis_index("c")
        sid = lax.axis_index("s")
        wid = sid * NC + cid
        r0 = sid * RN
        ebase = wid * EW
        gbase = soff + wid * EW

        def zero_row(i, carry):
            for k in range(NL):
                mbuf[0, i, pl.ds(k * 16, 16)] = jnp.zeros((16,), jnp.float32)
            return carry
        lax.fori_loop(0, RB, zero_row, 0)
        for j in range(NRB):
            pltpu.sync_copy(mbuf.at[0], agg_sh.at[pl.ds(r0 + j * RB, RB)])
        plsc.subcore_barrier()

        def issue_idx(c, slot):
            g0 = pl.multiple_of(gbase + c * C, 8)
            pltpu.async_copy(src_hbm.at[pl.ds(g0, C)],
                             idxr.at[slot, 0], isem.at[slot])
            pltpu.async_copy(dst_hbm.at[pl.ds(g0, C)],
                             idxr.at[slot, 1], isem.at[slot])

        def wait_idx(c, slot):
            g0 = pl.multiple_of(gbase + c * C, 8)
            pltpu.make_async_copy(src_hbm.at[pl.ds(g0, C)],
                                  idxr.at[slot, 0], isem.at[slot]).wait()
            pltpu.make_async_copy(dst_hbm.at[pl.ds(g0, C)],
                                  idxr.at[slot, 1], isem.at[slot]).wait()

        def issue_gather(c, b, slot):
            e0 = pl.multiple_of(ebase + c * C, 8)
            pltpu.async_copy(x_hbm.at[idxr.at[slot, 0]], rows.at[b],
                             gsem.at[b])
            pltpu.async_copy(z_hbm.at[pl.ds(e0, C)], zrows.at[b], zsem.at[b])

        for j in range(AHEAD):
            issue_idx(j, j)
        for j in range(NBUF):
            wait_idx(j, j)
            issue_gather(j, j, j)

        def chunk_body(c, carry):
            b = lax.rem(c, NBUF)
            sb = lax.rem(c, NSB)
            ib = lax.rem(c, NI)
            e0 = pl.multiple_of(ebase + c * C, 8)
            pltpu.make_async_copy(x_hbm.at[idxr.at[ib, 0]], rows.at[b],
                                  gsem.at[b]).wait()
            pltpu.make_async_copy(z_hbm.at[pl.ds(e0, C)], zrows.at[b],
                                  zsem.at[b]).wait()

            @pl.when(c >= NSB)
            def _wait_scatter():
                pltpu.make_async_copy(mbuf.at[sb], agg_sh.at[idxr.at[ib, 1]],
                                      ssem.at[sb]).wait()

            @plsc.parallel_loop(0, C, unroll=4)
            def row_body(i):
                for k in range(NL):
                    sl = pl.ds(k * 16, 16)
                    mbuf[sb, i, sl] = jnp.maximum(
                        rows[b, i, sl] + zrows[b, i, sl], 0.0)

            pltpu.async_copy(mbuf.at[sb], agg_sh.at[idxr.at[ib, 1]],
                             ssem.at[sb], add=True)

            @pl.when(c + AHEAD < NCH)
            def _prefetch_idx():
                issue_idx(c + AHEAD, lax.rem(c + AHEAD, NI))

            @pl.when(c + NBUF < NCH)
            def _prefetch_rows():
                cn = c + NBUF
                slot = lax.rem(cn, NI)
                wait_idx(cn, slot)
                issue_gather(cn, b, slot)
            return carry
        lax.fori_loop(0, NCH, chunk_body, 0)

        for j in range(NSB):
            c = NCH - NSB + j
            pltpu.make_async_copy(mbuf.at[c % NSB],
                                  agg_sh.at[idxr.at[c % NI, 1]],
                                  ssem.at[c % NSB]).wait()

        plsc.subcore_barrier()
        for j in range(NRB):
            pltpu.sync_copy(agg_sh.at[pl.ds(r0 + j * RB, RB)], mbuf.at[0])
            pltpu.sync_copy(mbuf.at[0], out_hbm.at[cid, pl.ds(r0 + j * RB, RB)])

    return agg_kernel(x, src, dst, z)


def _mlp(x, parts, W1, b1, W2, b2):
    N, D = x.shape
    NC = parts[0].shape[0]
    BN = 2000

    def body(x_ref, *rest):
        part_refs = rest[:-5]
        w1_ref, b1_ref, w2_ref, b2_ref, o_ref = rest[-5:]
        h = x_ref[...]
        for p_ref in part_refs:
            for c in range(NC):
                h = h + p_ref[c]
        h = jnp.maximum(
            jnp.dot(h, w1_ref[...], preferred_element_type=jnp.float32)
            + b1_ref[...], 0.0)
        o_ref[...] = jnp.dot(h, w2_ref[...],
                             preferred_element_type=jnp.float32) + b2_ref[...]

    part_specs = [pl.BlockSpec((NC, BN, D), lambda i: (0, i, 0))
                  for _ in parts]
    return pl.pallas_call(
        body,
        grid=(N // BN,),
        in_specs=[pl.BlockSpec((BN, D), lambda i: (i, 0))] + part_specs + [
            pl.BlockSpec((D, D), lambda i: (0, 0)),
            pl.BlockSpec((1, D), lambda i: (0, 0)),
            pl.BlockSpec((D, D), lambda i: (0, 0)),
            pl.BlockSpec((1, D), lambda i: (0, 0))],
        out_specs=pl.BlockSpec((BN, D), lambda i: (i, 0)),
        out_shape=jax.ShapeDtypeStruct((N, D), jnp.float32),
    )(x, *parts, W1, b1.reshape(1, D), W2, b2.reshape(1, D))


def kernel(x, edge_index, edge_attr, batch_idx, We, W1, b1, W2, b2):
    E = edge_index.shape[1]
    ES = E // _NSEG
    src, dst = _edge_repack(edge_index)
    edge_attr_t = edge_attr.T
    parts = []
    for s in range(_NSEG):
        z = _edge_matmul(edge_attr_t, We, s * ES, ES)
        parts.append(_sc_aggregate(x, src, dst, z, s * ES, ES))
    return _mlp(x, parts, W1, b1, W2, b2)
